# baseline (speedup 1.0000x reference)
import functools

import jax
import jax.numpy as jnp
from jax import lax
from jax.experimental import pallas as pl
from jax.experimental.pallas import tpu as pltpu
from jax.experimental.pallas import tpu_sc as plsc

NC, NS, L = 2, 16, 16


def _sc_gather_call(A, B, Zp, src, dst, *, chunk):
    n, d = A.shape
    e = src.shape[0]
    nw = NC * NS
    epw = e // nw
    nchunk = epw // chunk

    mesh = plsc.VectorSubcoreMesh(
        core_axis_name="c", subcore_axis_name="s",
        num_cores=NC, num_subcores=NS)

    dt = A.dtype

    def body(a_hbm, b_hbm, zp_hbm, src_hbm, dst_hbm,
             ad_out, bs_out, zs_out, zd_out,
             idx_s, idx_d, a_r, b_r, zs_r, zd_r, sem):
        cid = lax.axis_index("c")
        sid = lax.axis_index("s")
        base = (cid * NS + sid) * epw

        def chunk_body(c, _):
            off = base + c * chunk
            pltpu.sync_copy(src_hbm.at[pl.ds(off, chunk)], idx_s)
            pltpu.sync_copy(dst_hbm.at[pl.ds(off, chunk)], idx_d)
            cps = [
                pltpu.async_copy(a_hbm.at[idx_d], a_r, sem),
                pltpu.async_copy(b_hbm.at[idx_s], b_r, sem),
                pltpu.async_copy(zp_hbm.at[idx_s], zs_r, sem),
                pltpu.async_copy(zp_hbm.at[idx_d], zd_r, sem),
            ]
            for cp in cps:
                cp.wait()
            sl = pl.ds(off, chunk)
            cps = [
                pltpu.async_copy(a_r, ad_out.at[sl], sem),
                pltpu.async_copy(b_r, bs_out.at[sl], sem),
                pltpu.async_copy(zs_r, zs_out.at[sl], sem),
                pltpu.async_copy(zd_r, zd_out.at[sl], sem),
            ]
            for cp in cps:
                cp.wait()
            return 0

        lax.fori_loop(0, nchunk, chunk_body, 0)

    f = pl.kernel(
        body,
        out_type=[jax.ShapeDtypeStruct((e, d), dt),
                  jax.ShapeDtypeStruct((e, d), dt),
                  jax.ShapeDtypeStruct((e, L), jnp.float32),
                  jax.ShapeDtypeStruct((e, L), jnp.float32)],
        mesh=mesh,
        compiler_params=pltpu.CompilerParams(use_tc_tiling_on_sc=False),
        scratch_types=[
            pltpu.VMEM((chunk,), jnp.int32),
            pltpu.VMEM((chunk,), jnp.int32),
            pltpu.VMEM((chunk, d), dt),
            pltpu.VMEM((chunk, d), dt),
            pltpu.VMEM((chunk, L), jnp.float32),
            pltpu.VMEM((chunk, L), jnp.float32),
            pltpu.SemaphoreType.DMA,
        ],
    )
    return f(A, B, Zp, src, dst)


def _edge_call(Ad, Bs, Zs, Zd, edge_attr, wrow, WeE, wx, *, bn):
    e, d = Ad.shape
    de = edge_attr.shape[1]

    def body(ad, bs, zs, zd, ea, wr, we, wxr, m_o, c_o):
        relz = zs[...] - zd[...]
        d2 = jnp.sum(relz * relz, axis=1, keepdims=True)
        x = (ad[...].astype(jnp.float32) + bs[...].astype(jnp.float32)
             + d2 * wr[...]
             + jnp.dot(ea[...], we[...], preferred_element_type=jnp.float32))
        m = x * jax.nn.sigmoid(x)
        m_o[...] = m
        s = jnp.dot(m, wxr[...], preferred_element_type=jnp.float32)
        e3 = (lax.broadcasted_iota(jnp.int32, (1, L), 1) == 3).astype(jnp.float32)
        c_o[...] = relz * s + e3

    return pl.pallas_call(
        body,
        grid=(e // bn,),
        in_specs=[
            pl.BlockSpec((bn, d), lambda i: (i, 0)),
            pl.BlockSpec((bn, d), lambda i: (i, 0)),
            pl.BlockSpec((bn, L), lambda i: (i, 0)),
            pl.BlockSpec((bn, L), lambda i: (i, 0)),
            pl.BlockSpec((bn, de), lambda i: (i, 0)),
            pl.BlockSpec((1, d), lambda i: (0, 0)),
            pl.BlockSpec((de, d), lambda i: (0, 0)),
            pl.BlockSpec((d, 1), lambda i: (0, 0)),
        ],
        out_specs=[
            pl.BlockSpec((bn, d), lambda i: (i, 0)),
            pl.BlockSpec((bn, L), lambda i: (i, 0)),
        ],
        out_shape=[
            jax.ShapeDtypeStruct((e, d), jnp.float32),
            jax.ShapeDtypeStruct((e, L), jnp.float32),
        ],
    )(Ad, Bs, Zs, Zd, edge_attr, wrow, WeE, wx)


def _sc_scatter_call(M, C, dst, n, *, chunk):
    e, d = M.shape
    nw = NC * NS
    epw = e // nw
    nchunk = epw // chunk
    rows_pt = n // NS
    nzc, zrem = divmod(rows_pt, chunk)

    mesh = plsc.VectorSubcoreMesh(
        core_axis_name="c", subcore_axis_name="s",
        num_cores=NC, num_subcores=NS)

    def body(m_hbm, c_hbm, dst_hbm, aggm_out, aggx_out,
             idx_d, m_r, c_r, aggm_sh, aggx_sh, sem):
        cid = lax.axis_index("c")
        sid = lax.axis_index("s")
        base = (cid * NS + sid) * epw
        row0 = sid * rows_pt
        dsub = d // L

        zv = jnp.zeros((L,), jnp.float32)

        def zero_body(i, _):
            for j in range(dsub):
                m_r[i, pl.ds(j * L, L)] = zv
            c_r[i, :] = zv
            return 0

        lax.fori_loop(0, chunk, zero_body, 0)
        for k in range(nzc):
            pltpu.sync_copy(m_r, aggm_sh.at[pl.ds(row0 + k * chunk, chunk)])
            pltpu.sync_copy(c_r, aggx_sh.at[pl.ds(row0 + k * chunk, chunk)])
        if zrem:
            pltpu.sync_copy(m_r.at[pl.ds(0, zrem)],
                            aggm_sh.at[pl.ds(row0 + nzc * chunk, zrem)])
            pltpu.sync_copy(c_r.at[pl.ds(0, zrem)],
                            aggx_sh.at[pl.ds(row0 + nzc * chunk, zrem)])
        plsc.subcore_barrier()

        def chunk_body(c, _):
            off = base + c * chunk
            pltpu.sync_copy(dst_hbm.at[pl.ds(off, chunk)], idx_d)
            cps = [
                pltpu.async_copy(m_hbm.at[pl.ds(off, chunk)], m_r, sem),
                pltpu.async_copy(c_hbm.at[pl.ds(off, chunk)], c_r, sem),
            ]
            for cp in cps:
                cp.wait()
            pltpu.sync_copy(m_r, aggm_sh.at[idx_d], add=True)
            pltpu.sync_copy(c_r, aggx_sh.at[idx_d], add=True)
            return 0

        lax.fori_loop(0, nchunk, chunk_body, 0)
        plsc.subcore_barrier()

        pltpu.sync_copy(aggm_sh.at[pl.ds(row0, rows_pt)],
                        aggm_out.at[cid, sid])
        pltpu.sync_copy(aggx_sh.at[pl.ds(row0, rows_pt)],
                        aggx_out.at[cid, sid])

    f = pl.kernel(
        body,
        out_type=[jax.ShapeDtypeStruct((NC, NS, rows_pt, d), jnp.float32),
                  jax.ShapeDtypeStruct((NC, NS, rows_pt, L), jnp.float32)],
        mesh=mesh,
        compiler_params=pltpu.CompilerParams(use_tc_tiling_on_sc=False),
        scratch_types=[
            pltpu.VMEM((chunk,), jnp.int32),
            pltpu.VMEM((chunk, d), jnp.float32),
            pltpu.VMEM((chunk, L), jnp.float32),
            pltpu.VMEM_SHARED((n, d), jnp.float32),
            pltpu.VMEM_SHARED((n, L), jnp.float32),
            pltpu.SemaphoreType.DMA,
        ],
    )
    aggm, aggx = f(M, C, dst)
    return aggm.reshape(NC, n, d), aggx.reshape(NC, n, L)


def _pre_call(H, Z, WeA, WeB, be, *, bn):
    n, d = H.shape

    def body(h, z, wa, wb, b, a_o, b_o, zp_o):
        hv = h[...]
        a_o[...] = (jnp.dot(hv, wa[...], preferred_element_type=jnp.float32)
                    + b[...]).astype(jnp.bfloat16)
        b_o[...] = jnp.dot(hv, wb[...],
                           preferred_element_type=jnp.float32).astype(jnp.bfloat16)
        zv = z[...]
        zp_o[...] = jnp.concatenate(
            [zv, jnp.zeros((zv.shape[0], L - 3), jnp.float32)], axis=1)

    grid = (n // bn,)
    return pl.pallas_call(
        body,
        grid=grid,
        in_specs=[
            pl.BlockSpec((bn, d), lambda i: (i, 0)),
            pl.BlockSpec((bn, 3), lambda i: (i, 0)),
            pl.BlockSpec((d, d), lambda i: (0, 0)),
            pl.BlockSpec((d, d), lambda i: (0, 0)),
            pl.BlockSpec((1, d), lambda i: (0, 0)),
        ],
        out_specs=[
            pl.BlockSpec((bn, d), lambda i: (i, 0)),
            pl.BlockSpec((bn, d), lambda i: (i, 0)),
            pl.BlockSpec((bn, L), lambda i: (i, 0)),
        ],
        out_shape=[
            jax.ShapeDtypeStruct((n, d), jnp.bfloat16),
            jax.ShapeDtypeStruct((n, d), jnp.bfloat16),
            jax.ShapeDtypeStruct((n, L), jnp.float32),
        ],
    )(H, Z, WeA, WeB, be)


def _update_call(H, Zp, aggm, aggx, Wh, bh, WeA, WeB, be, *, bn, last):
    n, d = H.shape

    def body(h, zp, am, ax, wh, b, wa, wb, ben, h_o, zp_o, *ab_o):
        hv = h[...]
        agg = am[0] + am[1]
        upd = (jnp.dot(hv, wh[0], preferred_element_type=jnp.float32)
               + jnp.dot(agg, wh[1], preferred_element_type=jnp.float32)
               + b[...])
        hn = hv + upd * jax.nn.sigmoid(upd)
        h_o[...] = hn
        axv = ax[0] + ax[1]
        cnt = axv[:, 3:4]
        lmask = (lax.broadcasted_iota(jnp.int32, (1, L), 1) < 3).astype(jnp.float32)
        zp_o[...] = zp[...] + (axv * lmask) / (cnt + 1.0)
        if not last:
            ab_o[0][...] = (jnp.dot(hn, wa[...], preferred_element_type=jnp.float32)
                            + ben[...]).astype(jnp.bfloat16)
            ab_o[1][...] = jnp.dot(
                hn, wb[...],
                preferred_element_type=jnp.float32).astype(jnp.bfloat16)

    nb = n // bn
    out_specs = [pl.BlockSpec((bn, d), lambda i: (i, 0)),
                 pl.BlockSpec((bn, L), lambda i: (i, 0))]
    out_shape = [jax.ShapeDtypeStruct((n, d), jnp.float32),
                 jax.ShapeDtypeStruct((n, L), jnp.float32)]
    if not last:
        out_specs += [pl.BlockSpec((bn, d), lambda i: (i, 0)),
                      pl.BlockSpec((bn, d), lambda i: (i, 0))]
        out_shape += [jax.ShapeDtypeStruct((n, d), jnp.bfloat16),
                      jax.ShapeDtypeStruct((n, d), jnp.bfloat16)]
    return pl.pallas_call(
        body,
        grid=(nb,),
        in_specs=[
            pl.BlockSpec((bn, d), lambda i: (i, 0)),
            pl.BlockSpec((bn, L), lambda i: (i, 0)),
            pl.BlockSpec((NC, bn, d), lambda i: (0, i, 0)),
            pl.BlockSpec((NC, bn, L), lambda i: (0, i, 0)),
            pl.BlockSpec((2, d, d), lambda i: (0, 0, 0)),
            pl.BlockSpec((1, d), lambda i: (0, 0)),
            pl.BlockSpec((d, d), lambda i: (0, 0)),
            pl.BlockSpec((d, d), lambda i: (0, 0)),
            pl.BlockSpec((1, d), lambda i: (0, 0)),
        ],
        out_specs=out_specs,
        out_shape=out_shape,
    )(H, Zp, aggm, aggx, Wh, bh, WeA, WeB, be)


def _final_call(H, Zp, blk, maskf, *, bn, nseg):
    n, d = H.shape
    nb = n // bn

    def body(h, zp, b, mf, res_o, z_o):
        i = pl.program_id(0)
        mfv = mf[...]
        hm = h[...] * mfv
        onehot = (b[...] == lax.broadcasted_iota(jnp.int32, (1, nseg), 1)
                  ).astype(jnp.float32)
        part = lax.dot_general(onehot, hm, (((0,), (0,)), ((), ())),
                               preferred_element_type=jnp.float32)

        @pl.when(i == 0)
        def _():
            res_o[...] = part

        @pl.when(i > 0)
        def _():
            res_o[...] += part

        z_o[...] = zp[:, 0:3] * mfv

        @pl.when(i == nb - 1)
        def _():
            res = res_o[...]
            for _ in range(2):
                nrm = jnp.sqrt(jnp.sum(res * res, axis=1, keepdims=True))
                res = res / jnp.maximum(nrm, 1e-12)
            res_o[...] = res

    return pl.pallas_call(
        body,
        grid=(nb,),
        in_specs=[
            pl.BlockSpec((bn, d), lambda i: (i, 0)),
            pl.BlockSpec((bn, L), lambda i: (i, 0)),
            pl.BlockSpec((bn, 1), lambda i: (i, 0)),
            pl.BlockSpec((bn, 1), lambda i: (i, 0)),
        ],
        out_specs=[
            pl.BlockSpec((nseg, d), lambda i: (0, 0)),
            pl.BlockSpec((bn, 3), lambda i: (i, 0)),
        ],
        out_shape=[
            jax.ShapeDtypeStruct((nseg, d), jnp.float32),
            jax.ShapeDtypeStruct((n, 3), jnp.float32),
        ],
    )(H, Zp, blk, maskf)


def kernel(H, Z, block_id, batch_id, edges, edge_attr, mask_generate,
           mask_atoms, We, be, Wx, Wh, bh):
    n, d = H.shape
    nlayers = We.shape[0]
    nbk, lbk, na = mask_atoms.shape
    nseg = nbk * lbk
    e = edges.shape[1]
    src = edges[0]
    dst = edges[1]
    bn = n // 10

    A, B, Zp = _pre_call(H, Z, We[0, :d], We[0, d:2 * d], be[0:1], bn=bn)
    WhT = jnp.stack([Wh[:, :d, :], Wh[:, d:, :]], axis=1)
    for l in range(nlayers):
        Ad, Bs, Zs, Zd = _sc_gather_call(A, B, Zp, src, dst, chunk=400)
        M, C = _edge_call(Ad, Bs, Zs, Zd, edge_attr,
                          We[l, 2 * d:2 * d + 1], We[l, 2 * d + 1:],
                          Wx[l], bn=4000)
        aggm, aggx = _sc_scatter_call(M, C, dst, n, chunk=200)
        last = l == nlayers - 1
        nxt = 0 if last else l + 1
        outs = _update_call(H, Zp, aggm, aggx, WhT[l], bh[l:l + 1],
                            We[nxt, :d], We[nxt, d:2 * d], be[nxt:nxt + 1],
                            bn=bn, last=last)
        H, Zp = outs[0], outs[1]
        if not last:
            A, B = outs[2], outs[3]

    mask = jnp.where(mask_generate[:, :, None], True, mask_atoms)
    maskf = mask.reshape(-1, 1).astype(jnp.float32)
    res, z3 = _final_call(H, Zp, block_id.reshape(-1, 1).astype(jnp.int32),
                          maskf, bn=bn, nseg=nseg)
    H_out = res.reshape(nbk, lbk, d)
    Z_global = z3.reshape(nbk, lbk, na, 3)
    return (H_out, Z_global)


# speedup vs baseline: 1.4660x; 1.4660x over previous
import functools

import jax
import jax.numpy as jnp
from jax import lax
from jax.experimental import pallas as pl
from jax.experimental.pallas import tpu as pltpu
from jax.experimental.pallas import tpu_sc as plsc

NC, NS, L = 2, 16, 16


def _sc_gather_call(A, B, Zp, src, dst, *, chunk):
    n, d = A.shape
    e = src.shape[0]
    nw = NC * NS
    epw = e // nw
    nchunk = epw // chunk

    mesh = plsc.VectorSubcoreMesh(
        core_axis_name="c", subcore_axis_name="s",
        num_cores=NC, num_subcores=NS)

    dt = A.dtype

    def body(a_hbm, b_hbm, zp_hbm, src_hbm, dst_hbm,
             ad_out, bs_out, zs_out, zd_out,
             idx_s, idx_d, a_r, b_r, zs_r, zd_r, sem):
        cid = lax.axis_index("c")
        sid = lax.axis_index("s")
        base = (cid * NS + sid) * epw

        def chunk_body(c, _):
            off = base + c * chunk
            pltpu.sync_copy(src_hbm.at[pl.ds(off, chunk)], idx_s)
            pltpu.sync_copy(dst_hbm.at[pl.ds(off, chunk)], idx_d)
            cps = [
                pltpu.async_copy(a_hbm.at[idx_d], a_r, sem),
                pltpu.async_copy(b_hbm.at[idx_s], b_r, sem),
                pltpu.async_copy(zp_hbm.at[idx_s], zs_r, sem),
                pltpu.async_copy(zp_hbm.at[idx_d], zd_r, sem),
            ]
            for cp in cps:
                cp.wait()
            sl = pl.ds(off, chunk)
            cps = [
                pltpu.async_copy(a_r, ad_out.at[sl], sem),
                pltpu.async_copy(b_r, bs_out.at[sl], sem),
                pltpu.async_copy(zs_r, zs_out.at[sl], sem),
                pltpu.async_copy(zd_r, zd_out.at[sl], sem),
            ]
            for cp in cps:
                cp.wait()
            return 0

        lax.fori_loop(0, nchunk, chunk_body, 0)

    f = pl.kernel(
        body,
        out_type=[jax.ShapeDtypeStruct((e, d), dt),
                  jax.ShapeDtypeStruct((e, d), dt),
                  jax.ShapeDtypeStruct((e, L), jnp.float32),
                  jax.ShapeDtypeStruct((e, L), jnp.float32)],
        mesh=mesh,
        compiler_params=pltpu.CompilerParams(use_tc_tiling_on_sc=False),
        scratch_types=[
            pltpu.VMEM((chunk,), jnp.int32),
            pltpu.VMEM((chunk,), jnp.int32),
            pltpu.VMEM((chunk, d), dt),
            pltpu.VMEM((chunk, d), dt),
            pltpu.VMEM((chunk, L), jnp.float32),
            pltpu.VMEM((chunk, L), jnp.float32),
            pltpu.SemaphoreType.DMA,
        ],
    )
    return f(A, B, Zp, src, dst)


def _edge_call(Ad, Bs, Zs, Zd, edge_attr, wrow, WeE, wx, *, bn):
    e, d = Ad.shape
    de = edge_attr.shape[1]

    def body(ad, bs, zs, zd, ea, wr, we, wxr, m_o, c_o):
        relz = zs[...] - zd[...]
        d2 = jnp.sum(relz * relz, axis=1, keepdims=True)
        x = (ad[...] + bs[...] + d2 * wr[...]
             + jnp.dot(ea[...], we[...], preferred_element_type=jnp.float32))
        m = x * jax.nn.sigmoid(x)
        m_o[...] = m
        s = jnp.dot(m, wxr[...], preferred_element_type=jnp.float32)
        e3 = (lax.broadcasted_iota(jnp.int32, (1, L), 1) == 3).astype(jnp.float32)
        c_o[...] = relz * s + e3

    return pl.pallas_call(
        body,
        grid=(e // bn,),
        in_specs=[
            pl.BlockSpec((bn, d), lambda i: (i, 0)),
            pl.BlockSpec((bn, d), lambda i: (i, 0)),
            pl.BlockSpec((bn, L), lambda i: (i, 0)),
            pl.BlockSpec((bn, L), lambda i: (i, 0)),
            pl.BlockSpec((bn, de), lambda i: (i, 0)),
            pl.BlockSpec((1, d), lambda i: (0, 0)),
            pl.BlockSpec((de, d), lambda i: (0, 0)),
            pl.BlockSpec((d, 1), lambda i: (0, 0)),
        ],
        out_specs=[
            pl.BlockSpec((bn, d), lambda i: (i, 0)),
            pl.BlockSpec((bn, L), lambda i: (i, 0)),
        ],
        out_shape=[
            jax.ShapeDtypeStruct((e, d), jnp.float32),
            jax.ShapeDtypeStruct((e, L), jnp.float32),
        ],
    )(Ad, Bs, Zs, Zd, edge_attr, wrow, WeE, wx)


def _sc_scatter_call(M, C, dst, n, *, chunk):
    e, d = M.shape
    nw = NC * NS
    epw = e // nw
    nchunk = epw // chunk
    rows_pt = n // NS
    nzc, zrem = divmod(rows_pt, chunk)

    mesh = plsc.VectorSubcoreMesh(
        core_axis_name="c", subcore_axis_name="s",
        num_cores=NC, num_subcores=NS)

    def body(m_hbm, c_hbm, dst_hbm, aggm_out, aggx_out,
             idx_d, m_r, c_r, aggm_sh, aggx_sh, sem):
        cid = lax.axis_index("c")
        sid = lax.axis_index("s")
        base = (cid * NS + sid) * epw
        row0 = sid * rows_pt
        dsub = d // L

        zv = jnp.zeros((L,), jnp.float32)

        def zero_body(i, _):
            for j in range(dsub):
                m_r[i, pl.ds(j * L, L)] = zv
            c_r[i, :] = zv
            return 0

        lax.fori_loop(0, chunk, zero_body, 0)
        for k in range(nzc):
            pltpu.sync_copy(m_r, aggm_sh.at[pl.ds(row0 + k * chunk, chunk)])
            pltpu.sync_copy(c_r, aggx_sh.at[pl.ds(row0 + k * chunk, chunk)])
        if zrem:
            pltpu.sync_copy(m_r.at[pl.ds(0, zrem)],
                            aggm_sh.at[pl.ds(row0 + nzc * chunk, zrem)])
            pltpu.sync_copy(c_r.at[pl.ds(0, zrem)],
                            aggx_sh.at[pl.ds(row0 + nzc * chunk, zrem)])
        plsc.subcore_barrier()

        def chunk_body(c, _):
            off = base + c * chunk
            pltpu.sync_copy(dst_hbm.at[pl.ds(off, chunk)], idx_d)
            cps = [
                pltpu.async_copy(m_hbm.at[pl.ds(off, chunk)], m_r, sem),
                pltpu.async_copy(c_hbm.at[pl.ds(off, chunk)], c_r, sem),
            ]
            for cp in cps:
                cp.wait()
            pltpu.sync_copy(m_r, aggm_sh.at[idx_d], add=True)
            pltpu.sync_copy(c_r, aggx_sh.at[idx_d], add=True)
            return 0

        lax.fori_loop(0, nchunk, chunk_body, 0)
        plsc.subcore_barrier()

        pltpu.sync_copy(aggm_sh.at[pl.ds(row0, rows_pt)],
                        aggm_out.at[cid, sid])
        pltpu.sync_copy(aggx_sh.at[pl.ds(row0, rows_pt)],
                        aggx_out.at[cid, sid])

    f = pl.kernel(
        body,
        out_type=[jax.ShapeDtypeStruct((NC, NS, rows_pt, d), jnp.float32),
                  jax.ShapeDtypeStruct((NC, NS, rows_pt, L), jnp.float32)],
        mesh=mesh,
        compiler_params=pltpu.CompilerParams(use_tc_tiling_on_sc=False),
        scratch_types=[
            pltpu.VMEM((chunk,), jnp.int32),
            pltpu.VMEM((chunk, d), jnp.float32),
            pltpu.VMEM((chunk, L), jnp.float32),
            pltpu.VMEM_SHARED((n, d), jnp.float32),
            pltpu.VMEM_SHARED((n, L), jnp.float32),
            pltpu.SemaphoreType.DMA,
        ],
    )
    aggm, aggx = f(M, C, dst)
    return aggm.reshape(NC, n, d), aggx.reshape(NC, n, L)


def _pre_call(H, Z, WeA, WeB, be, *, bn):
    n, d = H.shape

    def body(h, z, wa, wb, b, a_o, b_o, zp_o):
        hv = h[...]
        a_o[...] = jnp.dot(hv, wa[...], preferred_element_type=jnp.float32) + b[...]
        b_o[...] = jnp.dot(hv, wb[...], preferred_element_type=jnp.float32)
        zv = z[...]
        zp_o[...] = jnp.concatenate(
            [zv, jnp.zeros((zv.shape[0], L - 3), jnp.float32)], axis=1)

    grid = (n // bn,)
    return pl.pallas_call(
        body,
        grid=grid,
        in_specs=[
            pl.BlockSpec((bn, d), lambda i: (i, 0)),
            pl.BlockSpec((bn, 3), lambda i: (i, 0)),
            pl.BlockSpec((d, d), lambda i: (0, 0)),
            pl.BlockSpec((d, d), lambda i: (0, 0)),
            pl.BlockSpec((1, d), lambda i: (0, 0)),
        ],
        out_specs=[
            pl.BlockSpec((bn, d), lambda i: (i, 0)),
            pl.BlockSpec((bn, d), lambda i: (i, 0)),
            pl.BlockSpec((bn, L), lambda i: (i, 0)),
        ],
        out_shape=[
            jax.ShapeDtypeStruct((n, d), jnp.float32),
            jax.ShapeDtypeStruct((n, d), jnp.float32),
            jax.ShapeDtypeStruct((n, L), jnp.float32),
        ],
    )(H, Z, WeA, WeB, be)


def _update_call(H, Zp, aggm, aggx, Wh, bh, WeA, WeB, be, *, bn, last):
    n, d = H.shape

    def body(h, zp, am, ax, wh, b, wa, wb, ben, h_o, zp_o, *ab_o):
        hv = h[...]
        agg = am[0] + am[1]
        upd = (jnp.dot(hv, wh[0], preferred_element_type=jnp.float32)
               + jnp.dot(agg, wh[1], preferred_element_type=jnp.float32)
               + b[...])
        hn = hv + upd * jax.nn.sigmoid(upd)
        h_o[...] = hn
        axv = ax[0] + ax[1]
        cnt = axv[:, 3:4]
        lmask = (lax.broadcasted_iota(jnp.int32, (1, L), 1) < 3).astype(jnp.float32)
        zp_o[...] = zp[...] + (axv * lmask) / (cnt + 1.0)
        if not last:
            ab_o[0][...] = jnp.dot(hn, wa[...], preferred_element_type=jnp.float32) + ben[...]
            ab_o[1][...] = jnp.dot(hn, wb[...], preferred_element_type=jnp.float32)

    nb = n // bn
    out_specs = [pl.BlockSpec((bn, d), lambda i: (i, 0)),
                 pl.BlockSpec((bn, L), lambda i: (i, 0))]
    out_shape = [jax.ShapeDtypeStruct((n, d), jnp.float32),
                 jax.ShapeDtypeStruct((n, L), jnp.float32)]
    if not last:
        out_specs += [pl.BlockSpec((bn, d), lambda i: (i, 0)),
                      pl.BlockSpec((bn, d), lambda i: (i, 0))]
        out_shape += [jax.ShapeDtypeStruct((n, d), jnp.float32),
                      jax.ShapeDtypeStruct((n, d), jnp.float32)]
    return pl.pallas_call(
        body,
        grid=(nb,),
        in_specs=[
            pl.BlockSpec((bn, d), lambda i: (i, 0)),
            pl.BlockSpec((bn, L), lambda i: (i, 0)),
            pl.BlockSpec((NC, bn, d), lambda i: (0, i, 0)),
            pl.BlockSpec((NC, bn, L), lambda i: (0, i, 0)),
            pl.BlockSpec((2, d, d), lambda i: (0, 0, 0)),
            pl.BlockSpec((1, d), lambda i: (0, 0)),
            pl.BlockSpec((d, d), lambda i: (0, 0)),
            pl.BlockSpec((d, d), lambda i: (0, 0)),
            pl.BlockSpec((1, d), lambda i: (0, 0)),
        ],
        out_specs=out_specs,
        out_shape=out_shape,
    )(H, Zp, aggm, aggx, Wh, bh, WeA, WeB, be)


def _final_call(H, Zp, blk, maskf, *, bn, nseg):
    n, d = H.shape
    nb = n // bn

    def body(h, zp, b, mf, res_o, z_o):
        i = pl.program_id(0)
        mfv = mf[...]
        hm = h[...] * mfv
        onehot = (b[...] == lax.broadcasted_iota(jnp.int32, (1, nseg), 1)
                  ).astype(jnp.float32)
        part = lax.dot_general(onehot, hm, (((0,), (0,)), ((), ())),
                               preferred_element_type=jnp.float32)

        @pl.when(i == 0)
        def _():
            res_o[...] = part

        @pl.when(i > 0)
        def _():
            res_o[...] += part

        z_o[...] = zp[:, 0:3] * mfv

        @pl.when(i == nb - 1)
        def _():
            res = res_o[...]
            for _ in range(2):
                nrm = jnp.sqrt(jnp.sum(res * res, axis=1, keepdims=True))
                res = res / jnp.maximum(nrm, 1e-12)
            res_o[...] = res

    return pl.pallas_call(
        body,
        grid=(nb,),
        in_specs=[
            pl.BlockSpec((bn, d), lambda i: (i, 0)),
            pl.BlockSpec((bn, L), lambda i: (i, 0)),
            pl.BlockSpec((bn, 1), lambda i: (i, 0)),
            pl.BlockSpec((bn, 1), lambda i: (i, 0)),
        ],
        out_specs=[
            pl.BlockSpec((nseg, d), lambda i: (0, 0)),
            pl.BlockSpec((bn, 3), lambda i: (i, 0)),
        ],
        out_shape=[
            jax.ShapeDtypeStruct((nseg, d), jnp.float32),
            jax.ShapeDtypeStruct((n, 3), jnp.float32),
        ],
    )(H, Zp, blk, maskf)


def kernel(H, Z, block_id, batch_id, edges, edge_attr, mask_generate,
           mask_atoms, We, be, Wx, Wh, bh):
    n, d = H.shape
    nlayers = We.shape[0]
    nbk, lbk, na = mask_atoms.shape
    nseg = nbk * lbk
    e = edges.shape[1]
    src = edges[0]
    dst = edges[1]
    bn = n // 10

    A, B, Zp = _pre_call(H, Z, We[0, :d], We[0, d:2 * d], be[0:1], bn=bn)
    WhT = jnp.stack([Wh[:, :d, :], Wh[:, d:, :]], axis=1)
    for l in range(nlayers):
        Ad, Bs, Zs, Zd = _sc_gather_call(A, B, Zp, src, dst, chunk=400)
        M, C = _edge_call(Ad, Bs, Zs, Zd, edge_attr,
                          We[l, 2 * d:2 * d + 1], We[l, 2 * d + 1:],
                          Wx[l], bn=4000)
        aggm, aggx = _sc_scatter_call(M, C, dst, n, chunk=200)
        last = l == nlayers - 1
        nxt = 0 if last else l + 1
        outs = _update_call(H, Zp, aggm, aggx, WhT[l], bh[l:l + 1],
                            We[nxt, :d], We[nxt, d:2 * d], be[nxt:nxt + 1],
                            bn=bn, last=last)
        H, Zp = outs[0], outs[1]
        if not last:
            A, B = outs[2], outs[3]

    mask = jnp.where(mask_generate[:, :, None], True, mask_atoms)
    maskf = mask.reshape(-1, 1).astype(jnp.float32)
    res, z3 = _final_call(H, Zp, block_id.reshape(-1, 1).astype(jnp.int32),
                          maskf, bn=bn, nseg=nseg)
    H_out = res.reshape(nbk, lbk, d)
    Z_global = z3.reshape(nbk, lbk, na, 3)
    return (H_out, Z_global)


# speedup vs baseline: 1.7651x; 1.2041x over previous
import functools

import jax
import jax.numpy as jnp
from jax import lax
from jax.experimental import pallas as pl
from jax.experimental.pallas import tpu as pltpu
from jax.experimental.pallas import tpu_sc as plsc

NC, NS, L = 2, 16, 16


def _sc_gather_call(A, B, Zp, Zn, src, dst, *, chunk):
    n, d = A.shape
    e = src.shape[0]
    nw = NC * NS
    epw = e // nw
    nchunk = epw // chunk

    mesh = plsc.VectorSubcoreMesh(
        core_axis_name="c", subcore_axis_name="s",
        num_cores=NC, num_subcores=NS)

    dt = A.dtype

    def body(a_hbm, b_hbm, zp_hbm, zn_hbm, src_hbm, dst_hbm,
             x_out, rel_out,
             idx_s, idx_d, x_r, z_r, sem):
        cid = lax.axis_index("c")
        sid = lax.axis_index("s")
        base = (cid * NS + sid) * epw

        def chunk_body(c, _):
            off = base + c * chunk
            pltpu.sync_copy(src_hbm.at[pl.ds(off, chunk)], idx_s)
            pltpu.sync_copy(dst_hbm.at[pl.ds(off, chunk)], idx_d)
            cps = [
                pltpu.async_copy(a_hbm.at[idx_d], x_r, sem),
                pltpu.async_copy(zp_hbm.at[idx_s], z_r, sem),
            ]
            for cp in cps:
                cp.wait()
            pltpu.sync_copy(b_hbm.at[idx_s], x_r, add=True)
            pltpu.sync_copy(zn_hbm.at[idx_d], z_r, add=True)
            sl = pl.ds(off, chunk)
            cps = [
                pltpu.async_copy(x_r, x_out.at[sl], sem),
                pltpu.async_copy(z_r, rel_out.at[sl], sem),
            ]
            for cp in cps:
                cp.wait()
            return 0

        lax.fori_loop(0, nchunk, chunk_body, 0)

    f = pl.kernel(
        body,
        out_type=[jax.ShapeDtypeStruct((e, d), dt),
                  jax.ShapeDtypeStruct((e, L), jnp.float32)],
        mesh=mesh,
        compiler_params=pltpu.CompilerParams(use_tc_tiling_on_sc=False),
        scratch_types=[
            pltpu.VMEM((chunk,), jnp.int32),
            pltpu.VMEM((chunk,), jnp.int32),
            pltpu.VMEM((chunk, d), dt),
            pltpu.VMEM((chunk, L), jnp.float32),
            pltpu.SemaphoreType.DMA,
        ],
    )
    return f(A, B, Zp, Zn, src, dst)


def _edge_call(X0, Rel, edge_attr, wrow, WeE, wx, *, bn):
    e, d = X0.shape
    de = edge_attr.shape[1]

    def body(x0, rel, ea, wr, we, wxr, m_o, c_o):
        relz = rel[...]
        d2 = jnp.sum(relz * relz, axis=1, keepdims=True)
        x = (x0[...] + d2 * wr[...]
             + jnp.dot(ea[...], we[...], preferred_element_type=jnp.float32))
        m = x * jax.nn.sigmoid(x)
        m_o[...] = m
        s = jnp.dot(m, wxr[...], preferred_element_type=jnp.float32)
        e3 = (lax.broadcasted_iota(jnp.int32, (1, L), 1) == 3).astype(jnp.float32)
        c_o[...] = relz * s + e3

    return pl.pallas_call(
        body,
        grid=(e // bn,),
        in_specs=[
            pl.BlockSpec((bn, d), lambda i: (i, 0)),
            pl.BlockSpec((bn, L), lambda i: (i, 0)),
            pl.BlockSpec((bn, de), lambda i: (i, 0)),
            pl.BlockSpec((1, d), lambda i: (0, 0)),
            pl.BlockSpec((de, d), lambda i: (0, 0)),
            pl.BlockSpec((d, 1), lambda i: (0, 0)),
        ],
        out_specs=[
            pl.BlockSpec((bn, d), lambda i: (i, 0)),
            pl.BlockSpec((bn, L), lambda i: (i, 0)),
        ],
        out_shape=[
            jax.ShapeDtypeStruct((e, d), jnp.float32),
            jax.ShapeDtypeStruct((e, L), jnp.float32),
        ],
    )(X0, Rel, edge_attr, wrow, WeE, wx)


def _sc_scatter_call(M, C, dst, n, *, chunk):
    e, d = M.shape
    nw = NC * NS
    epw = e // nw
    nchunk = epw // chunk
    rows_pt = n // NS
    nzc, zrem = divmod(rows_pt, chunk)

    mesh = plsc.VectorSubcoreMesh(
        core_axis_name="c", subcore_axis_name="s",
        num_cores=NC, num_subcores=NS)

    def body(m_hbm, c_hbm, dst_hbm, aggm_out, aggx_out,
             idx_d, m_r, c_r, aggm_sh, aggx_sh, sem):
        cid = lax.axis_index("c")
        sid = lax.axis_index("s")
        base = (cid * NS + sid) * epw
        row0 = sid * rows_pt
        dsub = d // L

        zv = jnp.zeros((L,), jnp.float32)

        def zero_body(i, _):
            for j in range(dsub):
                m_r[i, pl.ds(j * L, L)] = zv
            c_r[i, :] = zv
            return 0

        lax.fori_loop(0, chunk, zero_body, 0)
        for k in range(nzc):
            pltpu.sync_copy(m_r, aggm_sh.at[pl.ds(row0 + k * chunk, chunk)])
            pltpu.sync_copy(c_r, aggx_sh.at[pl.ds(row0 + k * chunk, chunk)])
        if zrem:
            pltpu.sync_copy(m_r.at[pl.ds(0, zrem)],
                            aggm_sh.at[pl.ds(row0 + nzc * chunk, zrem)])
            pltpu.sync_copy(c_r.at[pl.ds(0, zrem)],
                            aggx_sh.at[pl.ds(row0 + nzc * chunk, zrem)])
        plsc.subcore_barrier()

        def chunk_body(c, _):
            off = base + c * chunk
            pltpu.sync_copy(dst_hbm.at[pl.ds(off, chunk)], idx_d)
            cps = [
                pltpu.async_copy(m_hbm.at[pl.ds(off, chunk)], m_r, sem),
                pltpu.async_copy(c_hbm.at[pl.ds(off, chunk)], c_r, sem),
            ]
            for cp in cps:
                cp.wait()
            pltpu.sync_copy(m_r, aggm_sh.at[idx_d], add=True)
            pltpu.sync_copy(c_r, aggx_sh.at[idx_d], add=True)
            return 0

        lax.fori_loop(0, nchunk, chunk_body, 0)
        plsc.subcore_barrier()

        pltpu.sync_copy(aggm_sh.at[pl.ds(row0, rows_pt)],
                        aggm_out.at[cid, sid])
        pltpu.sync_copy(aggx_sh.at[pl.ds(row0, rows_pt)],
                        aggx_out.at[cid, sid])

    f = pl.kernel(
        body,
        out_type=[jax.ShapeDtypeStruct((NC, NS, rows_pt, d), jnp.float32),
                  jax.ShapeDtypeStruct((NC, NS, rows_pt, L), jnp.float32)],
        mesh=mesh,
        compiler_params=pltpu.CompilerParams(use_tc_tiling_on_sc=False),
        scratch_types=[
            pltpu.VMEM((chunk,), jnp.int32),
            pltpu.VMEM((chunk, d), jnp.float32),
            pltpu.VMEM((chunk, L), jnp.float32),
            pltpu.VMEM_SHARED((n, d), jnp.float32),
            pltpu.VMEM_SHARED((n, L), jnp.float32),
            pltpu.SemaphoreType.DMA,
        ],
    )
    aggm, aggx = f(M, C, dst)
    return aggm.reshape(NC, n, d), aggx.reshape(NC, n, L)


def _pre_call(H, Z, WeA, WeB, be, *, bn):
    n, d = H.shape

    def body(h, z, wa, wb, b, a_o, b_o, zp_o, zn_o):
        hv = h[...]
        a_o[...] = jnp.dot(hv, wa[...], preferred_element_type=jnp.float32) + b[...]
        b_o[...] = jnp.dot(hv, wb[...], preferred_element_type=jnp.float32)
        zv = z[...]
        zp = jnp.concatenate(
            [zv, jnp.zeros((zv.shape[0], L - 3), jnp.float32)], axis=1)
        zp_o[...] = zp
        zn_o[...] = -zp

    grid = (n // bn,)
    return pl.pallas_call(
        body,
        grid=grid,
        in_specs=[
            pl.BlockSpec((bn, d), lambda i: (i, 0)),
            pl.BlockSpec((bn, 3), lambda i: (i, 0)),
            pl.BlockSpec((d, d), lambda i: (0, 0)),
            pl.BlockSpec((d, d), lambda i: (0, 0)),
            pl.BlockSpec((1, d), lambda i: (0, 0)),
        ],
        out_specs=[
            pl.BlockSpec((bn, d), lambda i: (i, 0)),
            pl.BlockSpec((bn, d), lambda i: (i, 0)),
            pl.BlockSpec((bn, L), lambda i: (i, 0)),
            pl.BlockSpec((bn, L), lambda i: (i, 0)),
        ],
        out_shape=[
            jax.ShapeDtypeStruct((n, d), jnp.float32),
            jax.ShapeDtypeStruct((n, d), jnp.float32),
            jax.ShapeDtypeStruct((n, L), jnp.float32),
            jax.ShapeDtypeStruct((n, L), jnp.float32),
        ],
    )(H, Z, WeA, WeB, be)


def _update_call(H, Zp, aggm, aggx, Wh, bh, WeA, WeB, be, *, bn, last):
    n, d = H.shape

    def body(h, zp, am, ax, wh, b, wa, wb, ben, h_o, zp_o, *ab_o):
        hv = h[...]
        agg = am[0] + am[1]
        upd = (jnp.dot(hv, wh[0], preferred_element_type=jnp.float32)
               + jnp.dot(agg, wh[1], preferred_element_type=jnp.float32)
               + b[...])
        hn = hv + upd * jax.nn.sigmoid(upd)
        h_o[...] = hn
        axv = ax[0] + ax[1]
        cnt = axv[:, 3:4]
        lmask = (lax.broadcasted_iota(jnp.int32, (1, L), 1) < 3).astype(jnp.float32)
        zpn = zp[...] + (axv * lmask) / (cnt + 1.0)
        zp_o[...] = zpn
        if not last:
            ab_o[0][...] = jnp.dot(hn, wa[...], preferred_element_type=jnp.float32) + ben[...]
            ab_o[1][...] = jnp.dot(hn, wb[...], preferred_element_type=jnp.float32)
            ab_o[2][...] = -zpn

    nb = n // bn
    out_specs = [pl.BlockSpec((bn, d), lambda i: (i, 0)),
                 pl.BlockSpec((bn, L), lambda i: (i, 0))]
    out_shape = [jax.ShapeDtypeStruct((n, d), jnp.float32),
                 jax.ShapeDtypeStruct((n, L), jnp.float32)]
    if not last:
        out_specs += [pl.BlockSpec((bn, d), lambda i: (i, 0)),
                      pl.BlockSpec((bn, d), lambda i: (i, 0)),
                      pl.BlockSpec((bn, L), lambda i: (i, 0))]
        out_shape += [jax.ShapeDtypeStruct((n, d), jnp.float32),
                      jax.ShapeDtypeStruct((n, d), jnp.float32),
                      jax.ShapeDtypeStruct((n, L), jnp.float32)]
    return pl.pallas_call(
        body,
        grid=(nb,),
        in_specs=[
            pl.BlockSpec((bn, d), lambda i: (i, 0)),
            pl.BlockSpec((bn, L), lambda i: (i, 0)),
            pl.BlockSpec((NC, bn, d), lambda i: (0, i, 0)),
            pl.BlockSpec((NC, bn, L), lambda i: (0, i, 0)),
            pl.BlockSpec((2, d, d), lambda i: (0, 0, 0)),
            pl.BlockSpec((1, d), lambda i: (0, 0)),
            pl.BlockSpec((d, d), lambda i: (0, 0)),
            pl.BlockSpec((d, d), lambda i: (0, 0)),
            pl.BlockSpec((1, d), lambda i: (0, 0)),
        ],
        out_specs=out_specs,
        out_shape=out_shape,
    )(H, Zp, aggm, aggx, Wh, bh, WeA, WeB, be)


def _final_call(H, Zp, blk, maskf, *, bn, nseg):
    n, d = H.shape
    nb = n // bn

    def body(h, zp, b, mf, res_o, z_o):
        i = pl.program_id(0)
        mfv = mf[...]
        hm = h[...] * mfv
        onehot = (b[...] == lax.broadcasted_iota(jnp.int32, (1, nseg), 1)
                  ).astype(jnp.float32)
        part = lax.dot_general(onehot, hm, (((0,), (0,)), ((), ())),
                               preferred_element_type=jnp.float32)

        @pl.when(i == 0)
        def _():
            res_o[...] = part

        @pl.when(i > 0)
        def _():
            res_o[...] += part

        z_o[...] = zp[:, 0:3] * mfv

        @pl.when(i == nb - 1)
        def _():
            res = res_o[...]
            for _ in range(2):
                nrm = jnp.sqrt(jnp.sum(res * res, axis=1, keepdims=True))
                res = res / jnp.maximum(nrm, 1e-12)
            res_o[...] = res

    return pl.pallas_call(
        body,
        grid=(nb,),
        in_specs=[
            pl.BlockSpec((bn, d), lambda i: (i, 0)),
            pl.BlockSpec((bn, L), lambda i: (i, 0)),
            pl.BlockSpec((bn, 1), lambda i: (i, 0)),
            pl.BlockSpec((bn, 1), lambda i: (i, 0)),
        ],
        out_specs=[
            pl.BlockSpec((nseg, d), lambda i: (0, 0)),
            pl.BlockSpec((bn, 3), lambda i: (i, 0)),
        ],
        out_shape=[
            jax.ShapeDtypeStruct((nseg, d), jnp.float32),
            jax.ShapeDtypeStruct((n, 3), jnp.float32),
        ],
    )(H, Zp, blk, maskf)


def kernel(H, Z, block_id, batch_id, edges, edge_attr, mask_generate,
           mask_atoms, We, be, Wx, Wh, bh):
    n, d = H.shape
    nlayers = We.shape[0]
    nbk, lbk, na = mask_atoms.shape
    nseg = nbk * lbk
    e = edges.shape[1]
    src = edges[0]
    dst = edges[1]
    bn = n // 10

    A, B, Zp, Zn = _pre_call(H, Z, We[0, :d], We[0, d:2 * d], be[0:1], bn=bn)
    WhT = jnp.stack([Wh[:, :d, :], Wh[:, d:, :]], axis=1)
    for l in range(nlayers):
        X0, Rel = _sc_gather_call(A, B, Zp, Zn, src, dst, chunk=400)
        M, C = _edge_call(X0, Rel, edge_attr,
                          We[l, 2 * d:2 * d + 1], We[l, 2 * d + 1:],
                          Wx[l], bn=4000)
        aggm, aggx = _sc_scatter_call(M, C, dst, n, chunk=200)
        last = l == nlayers - 1
        nxt = 0 if last else l + 1
        outs = _update_call(H, Zp, aggm, aggx, WhT[l], bh[l:l + 1],
                            We[nxt, :d], We[nxt, d:2 * d], be[nxt:nxt + 1],
                            bn=bn, last=last)
        H, Zp = outs[0], outs[1]
        if not last:
            A, B, Zn = outs[2], outs[3], outs[4]

    mask = jnp.where(mask_generate[:, :, None], True, mask_atoms)
    maskf = mask.reshape(-1, 1).astype(jnp.float32)
    res, z3 = _final_call(H, Zp, block_id.reshape(-1, 1).astype(jnp.int32),
                          maskf, bn=bn, nseg=nseg)
    H_out = res.reshape(nbk, lbk, d)
    Z_global = z3.reshape(nbk, lbk, na, 3)
    return (H_out, Z_global)


# speedup vs baseline: 1.7926x; 1.0156x over previous
import functools

import jax
import jax.numpy as jnp
from jax import lax
from jax.experimental import pallas as pl
from jax.experimental.pallas import tpu as pltpu
from jax.experimental.pallas import tpu_sc as plsc

NC, NS, L = 2, 16, 16


def _sc_gather_call(A, B, Zp, Zn, src, dst, *, chunk, ebase, esz):
    n, d = A.shape
    nw = NC * NS
    epw = esz // nw
    nchunk = epw // chunk

    mesh = plsc.VectorSubcoreMesh(
        core_axis_name="c", subcore_axis_name="s",
        num_cores=NC, num_subcores=NS)

    dt = A.dtype

    def body(a_hbm, b_hbm, zp_hbm, zn_hbm, src_hbm, dst_hbm,
             x_out, rel_out,
             idx_s, idx_d, x_r, z_r, sem):
        cid = lax.axis_index("c")
        sid = lax.axis_index("s")
        base = (cid * NS + sid) * epw

        def chunk_body(c, _):
            off = base + c * chunk
            pltpu.sync_copy(src_hbm.at[pl.ds(ebase + off, chunk)], idx_s)
            pltpu.sync_copy(dst_hbm.at[pl.ds(ebase + off, chunk)], idx_d)
            cps = [
                pltpu.async_copy(a_hbm.at[idx_d], x_r, sem),
                pltpu.async_copy(zp_hbm.at[idx_s], z_r, sem),
            ]
            for cp in cps:
                cp.wait()
            pltpu.sync_copy(b_hbm.at[idx_s], x_r, add=True)
            pltpu.sync_copy(zn_hbm.at[idx_d], z_r, add=True)
            sl = pl.ds(off, chunk)
            cps = [
                pltpu.async_copy(x_r, x_out.at[sl], sem),
                pltpu.async_copy(z_r, rel_out.at[sl], sem),
            ]
            for cp in cps:
                cp.wait()
            return 0

        lax.fori_loop(0, nchunk, chunk_body, 0)

    f = pl.kernel(
        body,
        out_type=[jax.ShapeDtypeStruct((esz, d), dt),
                  jax.ShapeDtypeStruct((esz, L), jnp.float32)],
        mesh=mesh,
        compiler_params=pltpu.CompilerParams(use_tc_tiling_on_sc=False),
        scratch_types=[
            pltpu.VMEM((chunk,), jnp.int32),
            pltpu.VMEM((chunk,), jnp.int32),
            pltpu.VMEM((chunk, d), dt),
            pltpu.VMEM((chunk, L), jnp.float32),
            pltpu.SemaphoreType.DMA,
        ],
    )
    return f(A, B, Zp, Zn, src, dst)


def _edge_call(X0, Rel, edge_attr, wrow, WeE, wx, *, bn):
    e, d = X0.shape
    de = edge_attr.shape[1]

    def body(x0, rel, ea, wr, we, wxr, m_o, c_o):
        relz = rel[...]
        d2 = jnp.sum(relz * relz, axis=1, keepdims=True)
        x = (x0[...] + d2 * wr[...]
             + jnp.dot(ea[...], we[...], preferred_element_type=jnp.float32))
        m = x * jax.nn.sigmoid(x)
        m_o[...] = m
        s = jnp.dot(m, wxr[...], preferred_element_type=jnp.float32)
        e3 = (lax.broadcasted_iota(jnp.int32, (1, L), 1) == 3).astype(jnp.float32)
        c_o[...] = relz * s + e3

    return pl.pallas_call(
        body,
        grid=(e // bn,),
        in_specs=[
            pl.BlockSpec((bn, d), lambda i: (i, 0)),
            pl.BlockSpec((bn, L), lambda i: (i, 0)),
            pl.BlockSpec((bn, de), lambda i: (i, 0)),
            pl.BlockSpec((1, d), lambda i: (0, 0)),
            pl.BlockSpec((de, d), lambda i: (0, 0)),
            pl.BlockSpec((d, 1), lambda i: (0, 0)),
        ],
        out_specs=[
            pl.BlockSpec((bn, d), lambda i: (i, 0)),
            pl.BlockSpec((bn, L), lambda i: (i, 0)),
        ],
        out_shape=[
            jax.ShapeDtypeStruct((e, d), jnp.float32),
            jax.ShapeDtypeStruct((e, L), jnp.float32),
        ],
    )(X0, Rel, edge_attr, wrow, WeE, wx)


def _sc_scatter_call(M1, C1, M2, C2, dst, n, *, chunk):
    eh, d = M1.shape
    nw = NC * NS
    epw = eh // nw
    nchunk = epw // chunk
    rows_pt = n // NS
    nzc, zrem = divmod(rows_pt, chunk)

    mesh = plsc.VectorSubcoreMesh(
        core_axis_name="c", subcore_axis_name="s",
        num_cores=NC, num_subcores=NS)

    def body(m1_hbm, c1_hbm, m2_hbm, c2_hbm, dst_hbm, aggm_out, aggx_out,
             idx_d, m_r, c_r, aggm_sh, aggx_sh, sem):
        cid = lax.axis_index("c")
        sid = lax.axis_index("s")
        base = (cid * NS + sid) * epw
        row0 = sid * rows_pt
        dsub = d // L

        zv = jnp.zeros((L,), jnp.float32)

        def zero_body(i, _):
            for j in range(dsub):
                m_r[i, pl.ds(j * L, L)] = zv
            c_r[i, :] = zv
            return 0

        lax.fori_loop(0, chunk, zero_body, 0)
        for k in range(nzc):
            pltpu.sync_copy(m_r, aggm_sh.at[pl.ds(row0 + k * chunk, chunk)])
            pltpu.sync_copy(c_r, aggx_sh.at[pl.ds(row0 + k * chunk, chunk)])
        if zrem:
            pltpu.sync_copy(m_r.at[pl.ds(0, zrem)],
                            aggm_sh.at[pl.ds(row0 + nzc * chunk, zrem)])
            pltpu.sync_copy(c_r.at[pl.ds(0, zrem)],
                            aggx_sh.at[pl.ds(row0 + nzc * chunk, zrem)])
        plsc.subcore_barrier()

        for h, (m_hbm, c_hbm, goff) in enumerate(
                [(m1_hbm, c1_hbm, 0), (m2_hbm, c2_hbm, eh)]):
            def chunk_body(c, _):
                off = base + c * chunk
                pltpu.sync_copy(dst_hbm.at[pl.ds(goff + off, chunk)], idx_d)
                cps = [
                    pltpu.async_copy(m_hbm.at[pl.ds(off, chunk)], m_r, sem),
                    pltpu.async_copy(c_hbm.at[pl.ds(off, chunk)], c_r, sem),
                ]
                for cp in cps:
                    cp.wait()
                pltpu.sync_copy(m_r, aggm_sh.at[idx_d], add=True)
                pltpu.sync_copy(c_r, aggx_sh.at[idx_d], add=True)
                return 0

            lax.fori_loop(0, nchunk, chunk_body, 0)
        plsc.subcore_barrier()

        pltpu.sync_copy(aggm_sh.at[pl.ds(row0, rows_pt)],
                        aggm_out.at[cid, sid])
        pltpu.sync_copy(aggx_sh.at[pl.ds(row0, rows_pt)],
                        aggx_out.at[cid, sid])

    f = pl.kernel(
        body,
        out_type=[jax.ShapeDtypeStruct((NC, NS, rows_pt, d), jnp.float32),
                  jax.ShapeDtypeStruct((NC, NS, rows_pt, L), jnp.float32)],
        mesh=mesh,
        compiler_params=pltpu.CompilerParams(use_tc_tiling_on_sc=False),
        scratch_types=[
            pltpu.VMEM((chunk,), jnp.int32),
            pltpu.VMEM((chunk, d), jnp.float32),
            pltpu.VMEM((chunk, L), jnp.float32),
            pltpu.VMEM_SHARED((n, d), jnp.float32),
            pltpu.VMEM_SHARED((n, L), jnp.float32),
            pltpu.SemaphoreType.DMA,
        ],
    )
    aggm, aggx = f(M1, C1, M2, C2, dst)
    return aggm.reshape(NC, n, d), aggx.reshape(NC, n, L)


def _pre_call(H, Z, WeA, WeB, be, *, bn):
    n, d = H.shape

    def body(h, z, wa, wb, b, a_o, b_o, zp_o, zn_o):
        hv = h[...]
        a_o[...] = jnp.dot(hv, wa[...], preferred_element_type=jnp.float32) + b[...]
        b_o[...] = jnp.dot(hv, wb[...], preferred_element_type=jnp.float32)
        zv = z[...]
        zp = jnp.concatenate(
            [zv, jnp.zeros((zv.shape[0], L - 3), jnp.float32)], axis=1)
        zp_o[...] = zp
        zn_o[...] = -zp

    grid = (n // bn,)
    return pl.pallas_call(
        body,
        grid=grid,
        in_specs=[
            pl.BlockSpec((bn, d), lambda i: (i, 0)),
            pl.BlockSpec((bn, 3), lambda i: (i, 0)),
            pl.BlockSpec((d, d), lambda i: (0, 0)),
            pl.BlockSpec((d, d), lambda i: (0, 0)),
            pl.BlockSpec((1, d), lambda i: (0, 0)),
        ],
        out_specs=[
            pl.BlockSpec((bn, d), lambda i: (i, 0)),
            pl.BlockSpec((bn, d), lambda i: (i, 0)),
            pl.BlockSpec((bn, L), lambda i: (i, 0)),
            pl.BlockSpec((bn, L), lambda i: (i, 0)),
        ],
        out_shape=[
            jax.ShapeDtypeStruct((n, d), jnp.float32),
            jax.ShapeDtypeStruct((n, d), jnp.float32),
            jax.ShapeDtypeStruct((n, L), jnp.float32),
            jax.ShapeDtypeStruct((n, L), jnp.float32),
        ],
    )(H, Z, WeA, WeB, be)


def _update_call(H, Zp, aggm, aggx, Wh, bh, WeA, WeB, be, *, bn, last):
    n, d = H.shape

    def body(h, zp, am, ax, wh, b, wa, wb, ben, h_o, zp_o, *ab_o):
        hv = h[...]
        agg = am[0] + am[1]
        upd = (jnp.dot(hv, wh[0], preferred_element_type=jnp.float32)
               + jnp.dot(agg, wh[1], preferred_element_type=jnp.float32)
               + b[...])
        hn = hv + upd * jax.nn.sigmoid(upd)
        h_o[...] = hn
        axv = ax[0] + ax[1]
        cnt = axv[:, 3:4]
        lmask = (lax.broadcasted_iota(jnp.int32, (1, L), 1) < 3).astype(jnp.float32)
        zpn = zp[...] + (axv * lmask) / (cnt + 1.0)
        zp_o[...] = zpn
        if not last:
            ab_o[0][...] = jnp.dot(hn, wa[...], preferred_element_type=jnp.float32) + ben[...]
            ab_o[1][...] = jnp.dot(hn, wb[...], preferred_element_type=jnp.float32)
            ab_o[2][...] = -zpn

    nb = n // bn
    out_specs = [pl.BlockSpec((bn, d), lambda i: (i, 0)),
                 pl.BlockSpec((bn, L), lambda i: (i, 0))]
    out_shape = [jax.ShapeDtypeStruct((n, d), jnp.float32),
                 jax.ShapeDtypeStruct((n, L), jnp.float32)]
    if not last:
        out_specs += [pl.BlockSpec((bn, d), lambda i: (i, 0)),
                      pl.BlockSpec((bn, d), lambda i: (i, 0)),
                      pl.BlockSpec((bn, L), lambda i: (i, 0))]
        out_shape += [jax.ShapeDtypeStruct((n, d), jnp.float32),
                      jax.ShapeDtypeStruct((n, d), jnp.float32),
                      jax.ShapeDtypeStruct((n, L), jnp.float32)]
    return pl.pallas_call(
        body,
        grid=(nb,),
        in_specs=[
            pl.BlockSpec((bn, d), lambda i: (i, 0)),
            pl.BlockSpec((bn, L), lambda i: (i, 0)),
            pl.BlockSpec((NC, bn, d), lambda i: (0, i, 0)),
            pl.BlockSpec((NC, bn, L), lambda i: (0, i, 0)),
            pl.BlockSpec((2, d, d), lambda i: (0, 0, 0)),
            pl.BlockSpec((1, d), lambda i: (0, 0)),
            pl.BlockSpec((d, d), lambda i: (0, 0)),
            pl.BlockSpec((d, d), lambda i: (0, 0)),
            pl.BlockSpec((1, d), lambda i: (0, 0)),
        ],
        out_specs=out_specs,
        out_shape=out_shape,
    )(H, Zp, aggm, aggx, Wh, bh, WeA, WeB, be)


def _final_call(H, Zp, blk, maskf, *, bn, nseg):
    n, d = H.shape
    nb = n // bn

    def body(h, zp, b, mf, res_o, z_o):
        i = pl.program_id(0)
        mfv = mf[...]
        hm = h[...] * mfv
        onehot = (b[...] == lax.broadcasted_iota(jnp.int32, (1, nseg), 1)
                  ).astype(jnp.float32)
        part = lax.dot_general(onehot, hm, (((0,), (0,)), ((), ())),
                               preferred_element_type=jnp.float32)

        @pl.when(i == 0)
        def _():
            res_o[...] = part

        @pl.when(i > 0)
        def _():
            res_o[...] += part

        z_o[...] = zp[:, 0:3] * mfv

        @pl.when(i == nb - 1)
        def _():
            res = res_o[...]
            for _ in range(2):
                nrm = jnp.sqrt(jnp.sum(res * res, axis=1, keepdims=True))
                res = res / jnp.maximum(nrm, 1e-12)
            res_o[...] = res

    return pl.pallas_call(
        body,
        grid=(nb,),
        in_specs=[
            pl.BlockSpec((bn, d), lambda i: (i, 0)),
            pl.BlockSpec((bn, L), lambda i: (i, 0)),
            pl.BlockSpec((bn, 1), lambda i: (i, 0)),
            pl.BlockSpec((bn, 1), lambda i: (i, 0)),
        ],
        out_specs=[
            pl.BlockSpec((nseg, d), lambda i: (0, 0)),
            pl.BlockSpec((bn, 3), lambda i: (i, 0)),
        ],
        out_shape=[
            jax.ShapeDtypeStruct((nseg, d), jnp.float32),
            jax.ShapeDtypeStruct((n, 3), jnp.float32),
        ],
    )(H, Zp, blk, maskf)


def kernel(H, Z, block_id, batch_id, edges, edge_attr, mask_generate,
           mask_atoms, We, be, Wx, Wh, bh):
    n, d = H.shape
    nlayers = We.shape[0]
    nbk, lbk, na = mask_atoms.shape
    nseg = nbk * lbk
    e = edges.shape[1]
    src = edges[0]
    dst = edges[1]
    bn = n // 10

    A, B, Zp, Zn = _pre_call(H, Z, We[0, :d], We[0, d:2 * d], be[0:1], bn=bn)
    WhT = jnp.stack([Wh[:, :d, :], Wh[:, d:, :]], axis=1)
    eh = e // 2
    for l in range(nlayers):
        X1, R1 = _sc_gather_call(A, B, Zp, Zn, src, dst,
                                 chunk=200, ebase=0, esz=eh)
        X2, R2 = _sc_gather_call(A, B, Zp, Zn, src, dst,
                                 chunk=200, ebase=eh, esz=eh)
        M1, C1 = _edge_call(X1, R1, edge_attr[:eh],
                            We[l, 2 * d:2 * d + 1], We[l, 2 * d + 1:],
                            Wx[l], bn=4000)
        M2, C2 = _edge_call(X2, R2, edge_attr[eh:],
                            We[l, 2 * d:2 * d + 1], We[l, 2 * d + 1:],
                            Wx[l], bn=4000)
        aggm, aggx = _sc_scatter_call(M1, C1, M2, C2, dst, n, chunk=200)
        last = l == nlayers - 1
        nxt = 0 if last else l + 1
        outs = _update_call(H, Zp, aggm, aggx, WhT[l], bh[l:l + 1],
                            We[nxt, :d], We[nxt, d:2 * d], be[nxt:nxt + 1],
                            bn=bn, last=last)
        H, Zp = outs[0], outs[1]
        if not last:
            A, B, Zn = outs[2], outs[3], outs[4]

    mask = jnp.where(mask_generate[:, :, None], True, mask_atoms)
    maskf = mask.reshape(-1, 1).astype(jnp.float32)
    res, z3 = _final_call(H, Zp, block_id.reshape(-1, 1).astype(jnp.int32),
                          maskf, bn=bn, nseg=nseg)
    H_out = res.reshape(nbk, lbk, d)
    Z_global = z3.reshape(nbk, lbk, na, 3)
    return (H_out, Z_global)
